# Initial kernel scaffold; baseline (speedup 1.0000x reference)
import jax
import jax.numpy as jnp
from jax.experimental import pallas as pl


def kernel(x, edge_index, edge_attr, nbr, W_edge, W1, bn_gamma, bn_beta, W2):
    raise NotImplementedError("write your pallas kernel here")


# speedup vs baseline: 1.4223x; 1.4223x over previous
import functools

import jax
import jax.numpy as jnp
from jax import lax
from jax.experimental import pallas as pl
from jax.experimental.pallas import tpu as pltpu
from jax.experimental.pallas import tpu_sc as plsc

NC, NS, L = 2, 16, 16
NW = NC * NS
NV = 128 // L


def _wid():
    return lax.axis_index("s") * NC + lax.axis_index("c")


def _pipe2(nsteps, fire, wait, process):
    fire(0, 0)

    def body(jj, _):
        j0 = jj * 2
        fire(j0 + 1, 1)
        wait(j0, 0)
        process(j0, 0)

        @pl.when(j0 + 2 < nsteps)
        def _():
            fire(j0 + 2, 0)

        wait(j0 + 1, 1)
        process(j0 + 1, 1)
        return 0

    lax.fori_loop(0, nsteps // 2, body, 0, unroll=False)


def _mesh():
    return plsc.VectorSubcoreMesh(core_axis_name="c", subcore_axis_name="s")


def _xj_gather(x, src3, epad):
    cb = src3.shape[1]

    @functools.partial(
        pl.kernel,
        out_type=jax.ShapeDtypeStruct((epad, 128), jnp.float32),
        mesh=_mesh(),
        scratch_types=[
            pltpu.VMEM((cb, 128), jnp.int32),
            pltpu.VMEM((2, 128, 128), jnp.float32),
            pltpu.SemaphoreType.DMA,
            pltpu.SemaphoreType.DMA,
        ],
    )
    def k(x_hbm, src_hbm, xj_hbm, idx_v, rows_v, sem0, sem1):
        w = _wid()
        pltpu.sync_copy(src_hbm.at[w], idx_v)
        sems = (sem0, sem1)

        def fire(j, slot):
            pltpu.async_copy(x_hbm.at[idx_v.at[j]], rows_v.at[slot],
                             sems[slot])

        def wait(j, slot):
            pltpu.make_async_copy(x_hbm.at[idx_v.at[j]], rows_v.at[slot],
                                  sems[slot]).wait()

        def process(j, slot):
            pltpu.sync_copy(rows_v.at[slot],
                            xj_hbm.at[pl.ds((w * cb + j) * 128, 128)])

        _pipe2(cb, fire, wait, process)

    return k(x, src3)


def _msg_tc(xj, edge_attr, w_edge):
    e = edge_attr.shape[0]
    be = 2000
    grid = e // be

    def body(xj_ref, ea_ref, w_ref, o_ref):
        prod = lax.dot_general(ea_ref[...], w_ref[...],
                               (((1,), (1,)), ((), ())),
                               preferred_element_type=jnp.float32)
        o_ref[...] = jnp.maximum(xj_ref[...] + prod, 0.0) + 1e-07

    return pl.pallas_call(
        body,
        grid=(grid,),
        in_specs=[
            pl.BlockSpec((be, 128), lambda i: (i, 0)),
            pl.BlockSpec((be, 16), lambda i: (i, 0)),
            pl.BlockSpec((128, 16), lambda i: (0, 0)),
        ],
        out_specs=pl.BlockSpec((be, 128), lambda i: (i, 0)),
        out_shape=jax.ShapeDtypeStruct((e, 128), jnp.float32),
    )(xj, edge_attr, w_edge)


def _seg_max(msg, nbr3, dst, npad):
    nch = nbr3.shape[1]
    pb = nch * 128 // 32

    @functools.partial(
        pl.kernel,
        out_type=(jax.ShapeDtypeStruct((npad, 128), jnp.float32),
                  jax.ShapeDtypeStruct((NW, nch, 128), jnp.int32)),
        mesh=_mesh(),
        scratch_types=[
            pltpu.VMEM((nch, 128), jnp.int32),
            pltpu.VMEM((nch, 128), jnp.int32),
            pltpu.VMEM((2, 128, 128), jnp.float32),
            pltpu.VMEM((pb, 128), jnp.float32),
            pltpu.SemaphoreType.DMA,
            pltpu.SemaphoreType.DMA,
            pltpu.SemaphoreType.DMA,
        ],
    )
    def k(msg_hbm, nbr_hbm, dst_hbm, m_hbm, dnbr_hbm,
          nbr_v, dnbr_v, rows_v, m_v, semd, sem0, sem1):
        w = _wid()
        pltpu.sync_copy(nbr_hbm.at[w], nbr_v)

        def dfire(j, _):
            pltpu.async_copy(dst_hbm.at[nbr_v.at[j]], dnbr_v.at[j], semd)
            return 0

        lax.fori_loop(0, nch, dfire, 0, unroll=False)

        def ddrain(j, _):
            pltpu.make_async_copy(dst_hbm.at[nbr_v.at[j]], dnbr_v.at[j],
                                  semd).wait()
            return 0

        lax.fori_loop(0, nch, ddrain, 0, unroll=False)
        pltpu.sync_copy(dnbr_v, dnbr_hbm.at[w])

        sems = (sem0, sem1)

        def fire(j, slot):
            pltpu.async_copy(msg_hbm.at[nbr_v.at[j]], rows_v.at[slot],
                             sems[slot])

        def wait(j, slot):
            pltpu.make_async_copy(msg_hbm.at[nbr_v.at[j]], rows_v.at[slot],
                                  sems[slot]).wait()

        def process(j, slot):
            def node(i, _):
                r0 = i * 32
                acc0 = tuple(rows_v[slot, r0, pl.ds(v * L, L)]
                             for v in range(NV))

                def red(kk, acc):
                    return tuple(
                        jnp.maximum(acc[v],
                                    rows_v[slot, r0 + kk, pl.ds(v * L, L)])
                        for v in range(NV))

                acc = lax.fori_loop(1, 32, red, acc0, unroll=False)
                for v in range(NV):
                    m_v[j * 4 + i, pl.ds(v * L, L)] = acc[v]
                return 0

            lax.fori_loop(0, 4, node, 0, unroll=False)

        _pipe2(nch, fire, wait, process)
        pltpu.sync_copy(m_v, m_hbm.at[pl.ds(w * pb, pb)])

    return k(msg, nbr3, dst)


def _seg_sumexp(msg, m, nbr3, dnbr, npad):
    nch = nbr3.shape[1]
    pb = nch * 128 // 32

    @functools.partial(
        pl.kernel,
        out_type=jax.ShapeDtypeStruct((npad, 128), jnp.float32),
        mesh=_mesh(),
        scratch_types=[
            pltpu.VMEM((nch, 128), jnp.int32),
            pltpu.VMEM((nch, 128), jnp.int32),
            pltpu.VMEM((2, 128, 128), jnp.float32),
            pltpu.VMEM((2, 128, 128), jnp.float32),
            pltpu.VMEM((2, 4, 128), jnp.float32),
            pltpu.SemaphoreType.DMA,
            pltpu.SemaphoreType.DMA,
            pltpu.SemaphoreType.DMA,
        ],
    )
    def k(msg_hbm, m_hbm, nbr_hbm, dnbr_hbm, s_hbm,
          nbr_v, dnbr_v, rows_v, mrows_v, stage_v, sem0, sem1, semw):
        w = _wid()
        pltpu.sync_copy(nbr_hbm.at[w], nbr_v)
        pltpu.sync_copy(dnbr_hbm.at[w], dnbr_v)
        sems = (sem0, sem1)

        def fire(j, slot):
            pltpu.async_copy(msg_hbm.at[nbr_v.at[j]], rows_v.at[slot],
                             sems[slot])
            pltpu.async_copy(m_hbm.at[dnbr_v.at[j]], mrows_v.at[slot],
                             sems[slot])

        def wait(j, slot):
            pltpu.make_async_copy(msg_hbm.at[nbr_v.at[j]], rows_v.at[slot],
                                  sems[slot]).wait()
            pltpu.make_async_copy(m_hbm.at[dnbr_v.at[j]], mrows_v.at[slot],
                                  sems[slot]).wait()

        def wb(j, slot):
            return pltpu.make_async_copy(
                stage_v.at[slot], s_hbm.at[pl.ds(w * pb + j * 4, 4)], semw)

        def process(j, slot):
            @pl.when(j >= 2)
            def _():
                wb(j - 2, slot).wait()

            def node(i, _):
                r0 = i * 32
                acc0 = tuple(jnp.full((L,), 1e-16, jnp.float32)
                             for _v in range(NV))

                def red(kk, acc):
                    r = r0 + kk
                    return tuple(
                        acc[v] + jnp.exp(rows_v[slot, r, pl.ds(v * L, L)]
                                         - mrows_v[slot, r, pl.ds(v * L, L)])
                        for v in range(NV))

                acc = lax.fori_loop(0, 32, red, acc0, unroll=False)
                for v in range(NV):
                    stage_v[slot, i, pl.ds(v * L, L)] = acc[v]
                return 0

            lax.fori_loop(0, 4, node, 0, unroll=False)
            wb(j, slot).start()

        _pipe2(nch, fire, wait, process)
        wb(nch - 2, 0).wait()
        wb(nch - 1, 1).wait()

    return k(msg, m, nbr3, dnbr)


def _seg_wsum(msg, m, s, nbr3, dnbr, npad):
    nch = nbr3.shape[1]
    pb = nch * 128 // 32

    @functools.partial(
        pl.kernel,
        out_type=jax.ShapeDtypeStruct((npad, 128), jnp.float32),
        mesh=_mesh(),
        scratch_types=[
            pltpu.VMEM((nch, 128), jnp.int32),
            pltpu.VMEM((nch, 128), jnp.int32),
            pltpu.VMEM((2, 128, 128), jnp.float32),
            pltpu.VMEM((2, 128, 128), jnp.float32),
            pltpu.VMEM((2, 128, 128), jnp.float32),
            pltpu.VMEM((2, 4, 128), jnp.float32),
            pltpu.SemaphoreType.DMA,
            pltpu.SemaphoreType.DMA,
            pltpu.SemaphoreType.DMA,
        ],
    )
    def k(msg_hbm, m_hbm, s_hbm, nbr_hbm, dnbr_hbm, res_hbm,
          nbr_v, dnbr_v, rows_v, mrows_v, srows_v, stage_v, sem0, sem1, semw):
        w = _wid()
        pltpu.sync_copy(nbr_hbm.at[w], nbr_v)
        pltpu.sync_copy(dnbr_hbm.at[w], dnbr_v)
        sems = (sem0, sem1)

        def fire(j, slot):
            pltpu.async_copy(msg_hbm.at[nbr_v.at[j]], rows_v.at[slot],
                             sems[slot])
            pltpu.async_copy(m_hbm.at[dnbr_v.at[j]], mrows_v.at[slot],
                             sems[slot])
            pltpu.async_copy(s_hbm.at[dnbr_v.at[j]], srows_v.at[slot],
                             sems[slot])

        def wait(j, slot):
            pltpu.make_async_copy(msg_hbm.at[nbr_v.at[j]], rows_v.at[slot],
                                  sems[slot]).wait()
            pltpu.make_async_copy(m_hbm.at[dnbr_v.at[j]], mrows_v.at[slot],
                                  sems[slot]).wait()
            pltpu.make_async_copy(s_hbm.at[dnbr_v.at[j]], srows_v.at[slot],
                                  sems[slot]).wait()

        def wb(j, slot):
            return pltpu.make_async_copy(
                stage_v.at[slot], res_hbm.at[pl.ds(w * pb + j * 4, 4)], semw)

        def process(j, slot):
            @pl.when(j >= 2)
            def _():
                wb(j - 2, slot).wait()

            def node(i, _):
                r0 = i * 32
                acc0 = tuple(jnp.zeros((L,), jnp.float32) for _v in range(NV))

                def red(kk, acc):
                    r = r0 + kk
                    out = []
                    for v in range(NV):
                        mg = rows_v[slot, r, pl.ds(v * L, L)]
                        ex = jnp.exp(mg - mrows_v[slot, r, pl.ds(v * L, L)])
                        out.append(acc[v] + mg * ex
                                   / srows_v[slot, r, pl.ds(v * L, L)])
                    return tuple(out)

                acc = lax.fori_loop(0, 32, red, acc0, unroll=False)
                for v in range(NV):
                    stage_v[slot, i, pl.ds(v * L, L)] = acc[v]
                return 0

            lax.fori_loop(0, 4, node, 0, unroll=False)
            wb(j, slot).start()

        _pipe2(nch, fire, wait, process)
        wb(nch - 2, 0).wait()
        wb(nch - 1, 1).wait()

    return k(msg, m, s, nbr3, dnbr)


def _mlp_bn(res, x, w1, g2, b2, w2):
    n = x.shape[0]

    def body(res_ref, x_ref, w1_ref, g_ref, b_ref, w2_ref, o_ref):
        h0 = res_ref[...] + x_ref[...]
        h = lax.dot_general(h0, w1_ref[...], (((1,), (1,)), ((), ())),
                            preferred_element_type=jnp.float32)
        mu = jnp.mean(h, axis=0, keepdims=True)
        hc = h - mu
        var = jnp.mean(hc * hc, axis=0, keepdims=True)
        hn = hc * lax.rsqrt(var + 1e-5) * g_ref[...] + b_ref[...]
        hn = jnp.maximum(hn, 0.0)
        o_ref[...] = lax.dot_general(hn, w2_ref[...], (((1,), (1,)), ((), ())),
                                     preferred_element_type=jnp.float32)

    return pl.pallas_call(
        body,
        out_shape=jax.ShapeDtypeStruct((n, 128), jnp.float32),
    )(res, x, w1, g2, b2, w2)


def kernel(x, edge_index, edge_attr, nbr, W_edge, W1, bn_gamma, bn_beta, W2):
    n, d = x.shape
    e = edge_attr.shape[0]
    k = nbr.shape[1]

    src = edge_index[0]
    dst = edge_index[1]

    cb = -(-e // (NW * 128))
    cb += cb % 2
    epad = NW * cb * 128
    src3 = jnp.pad(src, (0, epad - e)).reshape(NW, cb, 128)

    npad = -(-n // NW) * NW
    pb = npad // NW
    while (pb * k) % 256 != 0:
        npad += NW
        pb = npad // NW
    nch = pb * k // 128
    nbr_p = jnp.pad(nbr, ((0, npad - n), (0, 0)))
    nbr3 = nbr_p.reshape(NW, nch, 128)

    xj = _xj_gather(x, src3, epad)
    msg = _msg_tc(xj, edge_attr, W_edge)
    m, dnbr = _seg_max(msg, nbr3, dst, npad)
    s = _seg_sumexp(msg, m, nbr3, dnbr, npad)
    res = _seg_wsum(msg, m, s, nbr3, dnbr, npad)
    return _mlp_bn(res[:n], x, W1, bn_gamma.reshape(1, -1),
                   bn_beta.reshape(1, -1), W2)
